# Initial kernel scaffold; baseline (speedup 1.0000x reference)
import functools

import jax
import jax.numpy as jnp
from jax import lax
from jax.experimental import pallas as pl
from jax.experimental.pallas import tpu as pltpu
from jax.experimental.pallas import tpu_sc as plsc

VOCAB = 1000000
EMBED_DIM = 64
BATCH = 16384
HIST = 50

_NC = 2
_NS = 16
_NW = _NC * _NS

_B = BATCH * HIST
_BPW = _B // _NW
_CHUNK = 128
_NCHUNK = _BPW // _CHUNK


def _gather_body(x_hbm, w_hbm, out_hbm, idx_v, buf0, buf1,
                 gsem0, gsem1, ssem0, ssem1):
    wid = lax.axis_index("s") * _NC + lax.axis_index("c")
    base = wid * _BPW

    pltpu.sync_copy(x_hbm.at[pl.ds(base, _BPW)], idx_v)

    bufs = (buf0, buf1)
    gsems = (gsem0, gsem1)
    ssems = (ssem0, ssem1)

    def start_gather(chunk, p):
        idx_slice = idx_v.at[pl.ds(chunk * _CHUNK, _CHUNK)]
        pltpu.async_copy(w_hbm.at[idx_slice], bufs[p], gsems[p])

    def start_scatter(chunk, p):
        dst = out_hbm.at[pl.ds(base + chunk * _CHUNK, _CHUNK)]
        pltpu.async_copy(bufs[p], dst, ssems[p])

    def wait_gather(p):
        pltpu.make_async_copy(w_hbm.at[idx_v.at[pl.ds(0, _CHUNK)]],
                              bufs[p], gsems[p]).wait()

    def wait_scatter(p):
        pltpu.make_async_copy(bufs[p], out_hbm.at[pl.ds(base, _CHUNK)],
                              ssems[p]).wait()

    start_gather(0, 0)
    start_gather(1, 1)

    def step(i, p):
        wait_gather(p)
        start_scatter(i, p)
        wait_scatter(p)
        start_gather(i + 2, p)

    def loop_body(it, carry):
        i = it * 2
        step(i, 0)
        step(i + 1, 1)
        return carry

    lax.fori_loop(0, (_NCHUNK - 2) // 2, loop_body, 0)

    for p, chunk in ((0, _NCHUNK - 2), (1, _NCHUNK - 1)):
        wait_gather(p)
        start_scatter(chunk, p)
        wait_scatter(p)


@jax.jit
def _gather_flat(x_flat, weight):
    mesh = plsc.VectorSubcoreMesh(core_axis_name="c", subcore_axis_name="s")
    kernel_fn = functools.partial(
        pl.kernel,
        mesh=mesh,
        out_type=jax.ShapeDtypeStruct((_B, EMBED_DIM), jnp.float32),
        scratch_types=[
            pltpu.VMEM((_BPW,), jnp.int32),
            pltpu.VMEM((_CHUNK, EMBED_DIM), jnp.float32),
            pltpu.VMEM((_CHUNK, EMBED_DIM), jnp.float32),
            pltpu.SemaphoreType.DMA,
            pltpu.SemaphoreType.DMA,
            pltpu.SemaphoreType.DMA,
            pltpu.SemaphoreType.DMA,
        ],
    )(_gather_body)
    return kernel_fn(x_flat, weight)


def kernel(x, weight):
    x_flat = x.reshape(-1).astype(jnp.int32)
    out = _gather_flat(x_flat, weight)
    return out.reshape(BATCH, HIST, EMBED_DIM)


# speedup vs baseline: 1.8386x; 1.8386x over previous
import functools

import jax
import jax.numpy as jnp
from jax import lax
from jax.experimental import pallas as pl
from jax.experimental.pallas import tpu as pltpu
from jax.experimental.pallas import tpu_sc as plsc

VOCAB = 1000000
EMBED_DIM = 64
BATCH = 16384
HIST = 50

_NC = 2
_NS = 16
_NW = _NC * _NS

_B = BATCH * HIST
_BPW = _B // _NW
_CHUNK = 128
_NCHUNK = _BPW // _CHUNK


def _gather_body(x_hbm, w_hbm, out_hbm, idx_v, buf0, buf1,
                 gsem0, gsem1, ssem0, ssem1):
    wid = lax.axis_index("s") * _NC + lax.axis_index("c")
    base = wid * _BPW

    pltpu.sync_copy(x_hbm.at[pl.ds(base, _BPW)], idx_v)

    bufs = (buf0, buf1)
    gsems = (gsem0, gsem1)
    ssems = (ssem0, ssem1)

    def start_gather(chunk, p):
        idx_slice = idx_v.at[pl.ds(chunk * _CHUNK, _CHUNK)]
        pltpu.async_copy(w_hbm.at[idx_slice], bufs[p], gsems[p])

    def start_scatter(chunk, p):
        dst = out_hbm.at[pl.ds(base + chunk * _CHUNK, _CHUNK)]
        pltpu.async_copy(bufs[p], dst, ssems[p])

    def wait_gather(p):
        pltpu.make_async_copy(w_hbm.at[idx_v.at[pl.ds(0, _CHUNK)]],
                              bufs[p], gsems[p]).wait()

    def wait_scatter(p):
        pltpu.make_async_copy(bufs[p], out_hbm.at[pl.ds(base, _CHUNK)],
                              ssems[p]).wait()

    start_gather(0, 0)
    start_gather(1, 1)

    def step(i, p):
        wait_gather(p)
        start_scatter(i, p)
        wait_scatter(p)
        start_gather(i + 2, p)

    def loop_body(it, carry):
        i = it * 2
        step(i, 0)
        step(i + 1, 1)
        return carry

    lax.fori_loop(0, (_NCHUNK - 2) // 2, loop_body, 0)

    for p, chunk in ((0, _NCHUNK - 2), (1, _NCHUNK - 1)):
        wait_gather(p)
        start_scatter(chunk, p)
        wait_scatter(p)


@jax.jit
def _gather_flat(x_flat, weight):
    mesh = plsc.VectorSubcoreMesh(core_axis_name="c", subcore_axis_name="s")
    kernel_fn = functools.partial(
        pl.kernel,
        mesh=mesh,
        compiler_params=pltpu.CompilerParams(use_tc_tiling_on_sc=False),
        out_type=jax.ShapeDtypeStruct((_B, EMBED_DIM), jnp.float32),
        scratch_types=[
            pltpu.VMEM((_BPW,), jnp.int32),
            pltpu.VMEM((_CHUNK, EMBED_DIM), jnp.float32),
            pltpu.VMEM((_CHUNK, EMBED_DIM), jnp.float32),
            pltpu.SemaphoreType.DMA,
            pltpu.SemaphoreType.DMA,
            pltpu.SemaphoreType.DMA,
            pltpu.SemaphoreType.DMA,
        ],
    )(_gather_body)
    return kernel_fn(x_flat, weight)


def kernel(x, weight):
    x_flat = x.reshape(-1).astype(jnp.int32)
    out = _gather_flat(x_flat, weight)
    return out.reshape(BATCH, HIST, EMBED_DIM)


# speedup vs baseline: 1.8799x; 1.0225x over previous
import functools

import jax
import jax.numpy as jnp
from jax import lax
from jax.experimental import pallas as pl
from jax.experimental.pallas import tpu as pltpu
from jax.experimental.pallas import tpu_sc as plsc

VOCAB = 1000000
EMBED_DIM = 64
BATCH = 16384
HIST = 50

_NC = 2
_NS = 16
_NW = _NC * _NS

_B = BATCH * HIST
_BPW = _B // _NW
_CHUNK = 128
_NCHUNK = _BPW // _CHUNK
_NBUF = 4


def _gather_body(x_hbm, w_hbm, out_hbm, idx_v, *scratch):
    bufs = scratch[:_NBUF]
    gsems = scratch[_NBUF:2 * _NBUF]
    ssems = scratch[2 * _NBUF:]

    wid = lax.axis_index("s") * _NC + lax.axis_index("c")
    base = wid * _BPW

    pltpu.sync_copy(x_hbm.at[pl.ds(base, _BPW)], idx_v)

    def start_gather(chunk, p):
        idx_slice = idx_v.at[pl.ds(chunk * _CHUNK, _CHUNK)]
        pltpu.async_copy(w_hbm.at[idx_slice], bufs[p], gsems[p])

    def start_scatter(chunk, p):
        dst = out_hbm.at[pl.ds(base + chunk * _CHUNK, _CHUNK)]
        pltpu.async_copy(bufs[p], dst, ssems[p])

    def wait_gather(p):
        pltpu.make_async_copy(w_hbm.at[idx_v.at[pl.ds(0, _CHUNK)]],
                              bufs[p], gsems[p]).wait()

    def wait_scatter(p):
        pltpu.make_async_copy(bufs[p], out_hbm.at[pl.ds(base, _CHUNK)],
                              ssems[p]).wait()

    for p in range(_NBUF):
        start_gather(p, p)

    def step(i, p):
        wait_gather(p)
        start_scatter(i, p)
        wait_scatter(p)
        start_gather(i + _NBUF, p)

    def loop_body(it, carry):
        i = it * _NBUF
        for p in range(_NBUF):
            step(i + p, p)
        return carry

    lax.fori_loop(0, (_NCHUNK - _NBUF) // _NBUF, loop_body, 0)

    for p in range(_NBUF):
        chunk = _NCHUNK - _NBUF + p
        wait_gather(p)
        start_scatter(chunk, p)
        wait_scatter(p)


@jax.jit
def _gather_flat(x_flat, weight):
    mesh = plsc.VectorSubcoreMesh(core_axis_name="c", subcore_axis_name="s")
    kernel_fn = functools.partial(
        pl.kernel,
        mesh=mesh,
        compiler_params=pltpu.CompilerParams(use_tc_tiling_on_sc=False),
        out_type=jax.ShapeDtypeStruct((_B, EMBED_DIM), jnp.float32),
        scratch_types=(
            [pltpu.VMEM((_BPW,), jnp.int32)]
            + [pltpu.VMEM((_CHUNK, EMBED_DIM), jnp.float32)] * _NBUF
            + [pltpu.SemaphoreType.DMA] * (2 * _NBUF)
        ),
    )(_gather_body)
    return kernel_fn(x_flat, weight)


def kernel(x, weight):
    x_flat = x.reshape(-1).astype(jnp.int32)
    out = _gather_flat(x_flat, weight)
    return out.reshape(BATCH, HIST, EMBED_DIM)


# speedup vs baseline: 1.8854x; 1.0029x over previous
import functools

import jax
import jax.numpy as jnp
from jax import lax
from jax.experimental import pallas as pl
from jax.experimental.pallas import tpu as pltpu
from jax.experimental.pallas import tpu_sc as plsc

VOCAB = 1000000
EMBED_DIM = 64
BATCH = 16384
HIST = 50

_NC = 2
_NS = 16
_NW = _NC * _NS

_B = BATCH * HIST
_BPW = _B // _NW
_CHUNK = 256
_NCHUNK = _BPW // _CHUNK
_NBUF = 4


def _gather_body(x_hbm, w_hbm, out_hbm, idx_v, *scratch):
    bufs = scratch[:_NBUF]
    gsems = scratch[_NBUF:2 * _NBUF]
    ssems = scratch[2 * _NBUF:]

    wid = lax.axis_index("s") * _NC + lax.axis_index("c")
    base = wid * _BPW

    pltpu.sync_copy(x_hbm.at[pl.ds(base, _BPW)], idx_v)

    def start_gather(chunk, p):
        idx_slice = idx_v.at[pl.ds(chunk * _CHUNK, _CHUNK)]
        pltpu.async_copy(w_hbm.at[idx_slice], bufs[p], gsems[p])

    def start_scatter(chunk, p):
        dst = out_hbm.at[pl.ds(base + chunk * _CHUNK, _CHUNK)]
        pltpu.async_copy(bufs[p], dst, ssems[p])

    def wait_gather(p):
        pltpu.make_async_copy(w_hbm.at[idx_v.at[pl.ds(0, _CHUNK)]],
                              bufs[p], gsems[p]).wait()

    def wait_scatter(p):
        pltpu.make_async_copy(bufs[p], out_hbm.at[pl.ds(base, _CHUNK)],
                              ssems[p]).wait()

    for p in range(_NBUF):
        start_gather(p, p)

    def step(i, p):
        wait_gather(p)
        start_scatter(i, p)
        wait_scatter(p)
        start_gather(i + _NBUF, p)

    def loop_body(it, carry):
        i = it * _NBUF
        for p in range(_NBUF):
            step(i + p, p)
        return carry

    lax.fori_loop(0, (_NCHUNK - _NBUF) // _NBUF, loop_body, 0)

    for p in range(_NBUF):
        chunk = _NCHUNK - _NBUF + p
        wait_gather(p)
        start_scatter(chunk, p)
        wait_scatter(p)


@jax.jit
def _gather_flat(x_flat, weight):
    mesh = plsc.VectorSubcoreMesh(core_axis_name="c", subcore_axis_name="s")
    kernel_fn = functools.partial(
        pl.kernel,
        mesh=mesh,
        compiler_params=pltpu.CompilerParams(use_tc_tiling_on_sc=False),
        out_type=jax.ShapeDtypeStruct((_B, EMBED_DIM), jnp.float32),
        scratch_types=(
            [pltpu.VMEM((_BPW,), jnp.int32)]
            + [pltpu.VMEM((_CHUNK, EMBED_DIM), jnp.float32)] * _NBUF
            + [pltpu.SemaphoreType.DMA] * (2 * _NBUF)
        ),
    )(_gather_body)
    return kernel_fn(x_flat, weight)


def kernel(x, weight):
    x_flat = x.reshape(-1).astype(jnp.int32)
    out = _gather_flat(x_flat, weight)
    return out.reshape(BATCH, HIST, EMBED_DIM)
